# Initial kernel scaffold; baseline (speedup 1.0000x reference)
import functools

import jax
import jax.numpy as jnp
from jax import lax
from jax.experimental import pallas as pl
from jax.experimental.pallas import tpu as pltpu
from jax.experimental.pallas import tpu_sc as plsc

N = 10000
E = 320000
DH = 128
DO = 64
EPS = 0.3

NC = 2
NS = 16
NW = NC * NS
NP = 10240
RPT = NP // NS
EC = E // NC
ET = E // NW
C = 80
NCHUNK = ET // C

_mesh = plsc.VectorSubcoreMesh(
    core_axis_name="c", subcore_axis_name="s", num_cores=NC, num_subcores=NS
)


@functools.partial(
    pl.kernel,
    out_type=jax.ShapeDtypeStruct((NC, NP), jnp.float32),
    mesh=_mesh,
    scratch_types=[
        pltpu.VMEM((C,), jnp.int32),
        pltpu.VMEM((C,), jnp.float32),
        pltpu.VMEM((RPT,), jnp.float32),
        pltpu.VMEM_SHARED((NP,), jnp.float32),
    ],
)
def _deg_kernel(dst_hbm, deg_out, dst_v, ones_v, zeros_v, deg_sh):
    cid = lax.axis_index("c")
    sid = lax.axis_index("s")

    def _fill(i, _):
        zeros_v[pl.ds(i * 16, 16)] = jnp.zeros((16,), jnp.float32)
        return 0

    lax.fori_loop(0, RPT // 16, _fill, 0)
    for i in range(C // 16):
        ones_v[pl.ds(i * 16, 16)] = jnp.ones((16,), jnp.float32)

    pltpu.sync_copy(zeros_v, deg_sh.at[pl.ds(sid * RPT, RPT)])
    plsc.subcore_barrier()

    base = cid * EC + sid * ET

    def _chunk(k, _):
        eo = pl.multiple_of(base + k * C, 8)
        pltpu.sync_copy(dst_hbm.at[pl.ds(eo, C)], dst_v)
        pltpu.sync_copy(ones_v, deg_sh.at[dst_v], add=True)
        return 0

    lax.fori_loop(0, NCHUNK, _chunk, 0)
    plsc.subcore_barrier()
    pltpu.sync_copy(
        deg_sh.at[pl.ds(sid * RPT, RPT)], deg_out.at[cid, pl.ds(sid * RPT, RPT)]
    )


@functools.partial(
    pl.kernel,
    out_type=jax.ShapeDtypeStruct((NC, NP, DH), jnp.float32),
    mesh=_mesh,
    scratch_types=[
        pltpu.VMEM((NP,), jnp.float32),
        pltpu.VMEM((NP,), jnp.float32),
        pltpu.VMEM((NP,), jnp.float32),
        pltpu.VMEM((C,), jnp.int32),
        pltpu.VMEM((C,), jnp.int32),
        pltpu.VMEM((C, DH), jnp.float32),
        pltpu.VMEM((C,), jnp.float32),
        pltpu.VMEM_SHARED((NP, DH), jnp.float32),
        pltpu.SemaphoreType.DMA,
    ],
)
def _edge_kernel(
    x_hbm, p_hbm, q_hbm, d_hbm, src_hbm, dst_hbm, z_out,
    p_v, q_v, d_v, src_v, dst_v, rows_v, coef_v, z_sh, sem,
):
    cid = lax.axis_index("c")
    sid = lax.axis_index("s")

    pltpu.sync_copy(p_hbm, p_v.at[pl.ds(0, N)])
    pltpu.sync_copy(q_hbm, q_v.at[pl.ds(0, N)])
    pltpu.sync_copy(d_hbm, d_v.at[pl.ds(0, N)])

    def _zrow(i, _):
        for j in range(DH // 16):
            rows_v[i, pl.ds(j * 16, 16)] = jnp.zeros((16,), jnp.float32)
        return 0

    lax.fori_loop(0, C, _zrow, 0)
    for i in range(RPT // C):
        pltpu.sync_copy(rows_v, z_sh.at[pl.ds(sid * RPT + i * C, C)])
    plsc.subcore_barrier()

    base = cid * EC + sid * ET

    def _chunk(k, _):
        eo = pl.multiple_of(base + k * C, 8)
        pltpu.sync_copy(src_hbm.at[pl.ds(eo, C)], src_v)
        pltpu.sync_copy(dst_hbm.at[pl.ds(eo, C)], dst_v)
        pltpu.async_copy(x_hbm.at[src_v], rows_v, sem).wait()

        def _gate(i, _):
            s = pl.ds(i * 16, 16)
            sv = src_v[s]
            dv = dst_v[s]
            t = plsc.load_gather(p_v, [dv]) + plsc.load_gather(q_v, [sv])
            e2 = jnp.exp(t + t)
            g = 1.0 - 2.0 / (e2 + 1.0)
            coef_v[s] = (
                g * plsc.load_gather(d_v, [dv]) * plsc.load_gather(d_v, [sv])
            )
            return 0

        lax.fori_loop(0, C // 16, _gate, 0)

        def _scale(e, _):
            cc = coef_v[e]
            for j in range(DH // 16):
                s = pl.ds(j * 16, 16)
                rows_v[e, s] = rows_v[e, s] * cc
            return 0

        lax.fori_loop(0, C, _scale, 0)
        pltpu.sync_copy(rows_v, z_sh.at[dst_v], add=True)
        return 0

    lax.fori_loop(0, NCHUNK, _chunk, 0)
    plsc.subcore_barrier()
    pltpu.sync_copy(
        z_sh.at[pl.ds(sid * RPT, RPT)], z_out.at[cid, pl.ds(sid * RPT, RPT)]
    )


_R = 1000
_GRID = N // _R


def _prologue_body(h_ref, w1_ref, b1_ref, g_ref, gb_ref, deg_ref, x_ref, pqd_ref):
    x = jnp.maximum(
        jnp.dot(h_ref[...], w1_ref[...], preferred_element_type=jnp.float32)
        + b1_ref[...],
        0.0,
    )
    x_ref[...] = x
    pq = jnp.dot(x, g_ref[...], preferred_element_type=jnp.float32) + gb_ref[...]
    d = lax.rsqrt(jnp.maximum(deg_ref[0, :] + deg_ref[1, :], 1.0))
    lane = lax.broadcasted_iota(jnp.int32, (_R, DH), 1)
    pqd_ref[...] = jnp.where(lane == 2, d[:, None], pq)


def _mid_body(raw_ref, z0_ref, z1_ref, g_ref, gb_ref, x_ref, pq_ref):
    x = EPS * raw_ref[...] + z0_ref[0] + z1_ref[0]
    x_ref[...] = x
    pq_ref[...] = (
        jnp.dot(x, g_ref[...], preferred_element_type=jnp.float32) + gb_ref[...]
    )


def _epilogue_body(raw_ref, z0_ref, z1_ref, w2_ref, b2_ref, out_ref):
    x = EPS * raw_ref[...] + z0_ref[0] + z1_ref[0]
    o = jnp.dot(x, w2_ref[...], preferred_element_type=jnp.float32) + b2_ref[...]
    m = jnp.max(o, axis=1, keepdims=True)
    s = o - m
    out_ref[...] = s - jnp.log(jnp.sum(jnp.exp(s), axis=1, keepdims=True))


def _row_spec():
    return pl.BlockSpec((_R, DH), lambda i: (i, 0))


def _full_spec(shape):
    return pl.BlockSpec(shape, lambda i: tuple(0 for _ in shape))


def _z_spec(core):
    return pl.BlockSpec((1, _R, DH), lambda i, c=core: (c, i, 0))


def _prologue(h, w1, b1row, gpad, gbrow, deg2):
    return pl.pallas_call(
        _prologue_body,
        grid=(_GRID,),
        in_specs=[
            _row_spec(),
            _full_spec((DH, DH)),
            _full_spec((1, DH)),
            _full_spec((DH, DH)),
            _full_spec((1, DH)),
            pl.BlockSpec((NC, _R), lambda i: (0, i)),
        ],
        out_specs=[_row_spec(), _row_spec()],
        out_shape=[
            jax.ShapeDtypeStruct((N, DH), jnp.float32),
            jax.ShapeDtypeStruct((N, DH), jnp.float32),
        ],
    )(h, w1, b1row, gpad, gbrow, deg2)


def _mid(raw, zp, gpad, gbrow):
    return pl.pallas_call(
        _mid_body,
        grid=(_GRID,),
        in_specs=[
            _row_spec(),
            _z_spec(0),
            _z_spec(1),
            _full_spec((DH, DH)),
            _full_spec((1, DH)),
        ],
        out_specs=[_row_spec(), _row_spec()],
        out_shape=[
            jax.ShapeDtypeStruct((N, DH), jnp.float32),
            jax.ShapeDtypeStruct((N, DH), jnp.float32),
        ],
    )(raw, zp, zp, gpad, gbrow)


def _epilogue(raw, zp, w2, b2row):
    return pl.pallas_call(
        _epilogue_body,
        grid=(_GRID,),
        in_specs=[
            _row_spec(),
            _z_spec(0),
            _z_spec(1),
            _full_spec((DH, DO)),
            _full_spec((1, DO)),
        ],
        out_specs=pl.BlockSpec((_R, DO), lambda i: (i, 0)),
        out_shape=jax.ShapeDtypeStruct((N, DO), jnp.float32),
    )(raw, zp, zp, w2, b2row)


def _gate_pack(gate_W, gate_b):
    g = jnp.zeros((DH, DH), jnp.float32)
    g = g.at[:, 0].set(gate_W[:DH, 0]).at[:, 1].set(gate_W[DH:, 0])
    b = jnp.zeros((1, DH), jnp.float32).at[0, 0].set(gate_b[0])
    return g, b


def kernel(h, edge_index, t1_W, t1_b, gate_W0, gate_b0, gate_W1, gate_b1, t2_W, t2_b):
    src = edge_index[0]
    dst = edge_index[1]

    deg2 = _deg_kernel(dst)

    g0, gb0 = _gate_pack(gate_W0, gate_b0)
    g1, gb1 = _gate_pack(gate_W1, gate_b1)
    b1row = t1_b.reshape(1, DH)
    b2row = t2_b.reshape(1, DO)

    raw, pqd0 = _prologue(h, t1_W, b1row, g0, gb0, deg2)
    d = pqd0[:, 2]

    zp0 = _edge_kernel(raw, pqd0[:, 0], pqd0[:, 1], d, src, dst)
    x1, pq1 = _mid(raw, zp0, g1, gb1)
    zp1 = _edge_kernel(x1, pq1[:, 0], pq1[:, 1], d, src, dst)
    return _epilogue(raw, zp1, t2_W, b2row)


# speedup vs baseline: 10.6221x; 10.6221x over previous
import functools

import jax
import jax.numpy as jnp
from jax import lax
from jax.experimental import pallas as pl
from jax.experimental.pallas import tpu as pltpu
from jax.experimental.pallas import tpu_sc as plsc

N = 10000
E = 320000
DH = 128
DO = 64
EPS = 0.3

NC = 2
NS = 16
NW = NC * NS
NP = 10240
RPT = NP // NS
EC = E // NC
ET = E // NW
C = 80
NCHUNK = ET // C

_mesh = plsc.VectorSubcoreMesh(
    core_axis_name="c", subcore_axis_name="s", num_cores=NC, num_subcores=NS
)


@functools.partial(
    pl.kernel,
    out_type=jax.ShapeDtypeStruct((NC, NP), jnp.float32),
    mesh=_mesh,
    scratch_types=[
        pltpu.VMEM((C,), jnp.int32),
        pltpu.VMEM((C,), jnp.float32),
        pltpu.VMEM((RPT,), jnp.float32),
        pltpu.VMEM_SHARED((NP,), jnp.float32),
    ],
    compiler_params=pltpu.CompilerParams(needs_layout_passes=False),
)
def _deg_kernel(dst_hbm, deg_out, dst_v, ones_v, zeros_v, deg_sh):
    cid = lax.axis_index("c")
    sid = lax.axis_index("s")

    def _fill(i, _):
        zeros_v[pl.ds(i * 16, 16)] = jnp.zeros((16,), jnp.float32)
        return 0

    lax.fori_loop(0, RPT // 16, _fill, 0)
    for i in range(C // 16):
        ones_v[pl.ds(i * 16, 16)] = jnp.ones((16,), jnp.float32)

    pltpu.sync_copy(zeros_v, deg_sh.at[pl.ds(sid * RPT, RPT)])
    plsc.subcore_barrier()

    base = cid * EC + sid * ET

    def _chunk(k, _):
        eo = pl.multiple_of(base + k * C, 8)
        pltpu.sync_copy(dst_hbm.at[pl.ds(eo, C)], dst_v)
        pltpu.sync_copy(ones_v, deg_sh.at[dst_v], add=True)
        return 0

    lax.fori_loop(0, NCHUNK, _chunk, 0)
    plsc.subcore_barrier()
    pltpu.sync_copy(
        deg_sh.at[pl.ds(sid * RPT, RPT)], deg_out.at[cid, pl.ds(sid * RPT, RPT)]
    )


@functools.partial(
    pl.kernel,
    out_type=jax.ShapeDtypeStruct((NC, NP, DH), jnp.float32),
    mesh=_mesh,
    scratch_types=[
        pltpu.VMEM((NP,), jnp.float32),
        pltpu.VMEM((NP,), jnp.float32),
        pltpu.VMEM((NP,), jnp.float32),
        pltpu.VMEM((C,), jnp.int32),
        pltpu.VMEM((C,), jnp.int32),
        pltpu.VMEM((C, DH), jnp.float32),
        pltpu.VMEM((C,), jnp.float32),
        pltpu.VMEM_SHARED((NP, DH), jnp.float32),
        pltpu.SemaphoreType.DMA,
    ],
    compiler_params=pltpu.CompilerParams(needs_layout_passes=False),
)
def _edge_kernel(
    x_hbm, p_hbm, q_hbm, deg_hbm, src_hbm, dst_hbm, z_out,
    p_v, q_v, d_v, src_v, dst_v, rows_v, coef_v, z_sh, sem,
):
    cid = lax.axis_index("c")
    sid = lax.axis_index("s")

    pltpu.sync_copy(deg_hbm.at[0], d_v)
    pltpu.sync_copy(deg_hbm.at[1], p_v)

    def _dcalc(i, _):
        s = pl.ds(i * 16, 16)
        x = jnp.maximum(d_v[s] + p_v[s], 1.0)
        ii = 0x5F3759DF - lax.shift_right_arithmetic(
            lax.bitcast_convert_type(x, jnp.int32), 1
        )
        y = lax.bitcast_convert_type(ii, jnp.float32)
        for _unused in range(3):
            y = y * (1.5 - 0.5 * x * y * y)
        d_v[s] = y
        return 0

    lax.fori_loop(0, NP // 16, _dcalc, 0)

    pltpu.sync_copy(p_hbm, p_v.at[pl.ds(0, N)])
    pltpu.sync_copy(q_hbm, q_v.at[pl.ds(0, N)])

    def _zrow(i, _):
        for j in range(DH // 16):
            rows_v[i, pl.ds(j * 16, 16)] = jnp.zeros((16,), jnp.float32)
        return 0

    lax.fori_loop(0, C, _zrow, 0)
    for i in range(RPT // C):
        pltpu.sync_copy(rows_v, z_sh.at[pl.ds(sid * RPT + i * C, C)])
    plsc.subcore_barrier()

    base = cid * EC + sid * ET

    def _chunk(k, _):
        eo = pl.multiple_of(base + k * C, 8)
        pltpu.sync_copy(src_hbm.at[pl.ds(eo, C)], src_v)
        pltpu.sync_copy(dst_hbm.at[pl.ds(eo, C)], dst_v)
        pltpu.async_copy(x_hbm.at[src_v], rows_v, sem).wait()

        def _gate(i, _):
            s = pl.ds(i * 16, 16)
            sv = src_v[s]
            dv = dst_v[s]
            t = plsc.load_gather(p_v, [dv]) + plsc.load_gather(q_v, [sv])
            e2 = jnp.exp(t + t)
            g = 1.0 - 2.0 / (e2 + 1.0)
            coef_v[s] = (
                g * plsc.load_gather(d_v, [dv]) * plsc.load_gather(d_v, [sv])
            )
            return 0

        lax.fori_loop(0, C // 16, _gate, 0)

        def _scale(i, _):
            cvec = coef_v[pl.ds(i * 16, 16)]
            for l in range(16):
                e = i * 16 + l
                cc = cvec[l]
                for j in range(DH // 16):
                    s = pl.ds(j * 16, 16)
                    rows_v[e, s] = rows_v[e, s] * cc
            return 0

        lax.fori_loop(0, C // 16, _scale, 0)
        pltpu.sync_copy(rows_v, z_sh.at[dst_v], add=True)
        return 0

    lax.fori_loop(0, NCHUNK, _chunk, 0)
    plsc.subcore_barrier()
    pltpu.sync_copy(
        z_sh.at[pl.ds(sid * RPT, RPT)], z_out.at[cid, pl.ds(sid * RPT, RPT)]
    )


_R = 1000
_GRID = N // _R


def _prologue_body(h_ref, w1_ref, b1_ref, g_ref, gb_ref, x_ref, pq_ref):
    x = jnp.maximum(
        jnp.dot(h_ref[...], w1_ref[...], preferred_element_type=jnp.float32)
        + b1_ref[...],
        0.0,
    )
    x_ref[...] = x
    pq_ref[...] = (
        jnp.dot(x, g_ref[...], preferred_element_type=jnp.float32) + gb_ref[...]
    )


def _mid_body(raw_ref, z0_ref, z1_ref, g_ref, gb_ref, x_ref, pq_ref):
    x = EPS * raw_ref[...] + z0_ref[0] + z1_ref[0]
    x_ref[...] = x
    pq_ref[...] = (
        jnp.dot(x, g_ref[...], preferred_element_type=jnp.float32) + gb_ref[...]
    )


def _epilogue_body(raw_ref, z0_ref, z1_ref, w2_ref, b2_ref, out_ref):
    x = EPS * raw_ref[...] + z0_ref[0] + z1_ref[0]
    o = jnp.dot(x, w2_ref[...], preferred_element_type=jnp.float32) + b2_ref[...]
    m = jnp.max(o, axis=1, keepdims=True)
    s = o - m
    out_ref[...] = s - jnp.log(jnp.sum(jnp.exp(s), axis=1, keepdims=True))


def _row_spec():
    return pl.BlockSpec((_R, DH), lambda i: (i, 0))


def _full_spec(shape):
    return pl.BlockSpec(shape, lambda i: tuple(0 for _ in shape))


def _z_spec(core):
    return pl.BlockSpec((1, _R, DH), lambda i, c=core: (c, i, 0))


def _prologue(h, w1, b1row, gpad, gbrow):
    return pl.pallas_call(
        _prologue_body,
        grid=(_GRID,),
        in_specs=[
            _row_spec(),
            _full_spec((DH, DH)),
            _full_spec((1, DH)),
            _full_spec((DH, DH)),
            _full_spec((1, DH)),
        ],
        out_specs=[_row_spec(), _row_spec()],
        out_shape=[
            jax.ShapeDtypeStruct((N, DH), jnp.float32),
            jax.ShapeDtypeStruct((N, DH), jnp.float32),
        ],
    )(h, w1, b1row, gpad, gbrow)


def _mid(raw, zp, gpad, gbrow):
    return pl.pallas_call(
        _mid_body,
        grid=(_GRID,),
        in_specs=[
            _row_spec(),
            _z_spec(0),
            _z_spec(1),
            _full_spec((DH, DH)),
            _full_spec((1, DH)),
        ],
        out_specs=[_row_spec(), _row_spec()],
        out_shape=[
            jax.ShapeDtypeStruct((N, DH), jnp.float32),
            jax.ShapeDtypeStruct((N, DH), jnp.float32),
        ],
    )(raw, zp, zp, gpad, gbrow)


def _epilogue(raw, zp, w2, b2row):
    return pl.pallas_call(
        _epilogue_body,
        grid=(_GRID,),
        in_specs=[
            _row_spec(),
            _z_spec(0),
            _z_spec(1),
            _full_spec((DH, DO)),
            _full_spec((1, DO)),
        ],
        out_specs=pl.BlockSpec((_R, DO), lambda i: (i, 0)),
        out_shape=jax.ShapeDtypeStruct((N, DO), jnp.float32),
    )(raw, zp, zp, w2, b2row)


def _gate_pack(gate_W, gate_b):
    g = jnp.zeros((DH, DH), jnp.float32)
    g = g.at[:, 0].set(gate_W[:DH, 0]).at[:, 1].set(gate_W[DH:, 0])
    b = jnp.zeros((1, DH), jnp.float32).at[0, 0].set(gate_b[0])
    return g, b


def kernel(h, edge_index, t1_W, t1_b, gate_W0, gate_b0, gate_W1, gate_b1, t2_W, t2_b):
    src = edge_index[0]
    dst = edge_index[1]

    deg2 = _deg_kernel(dst)

    g0, gb0 = _gate_pack(gate_W0, gate_b0)
    g1, gb1 = _gate_pack(gate_W1, gate_b1)
    b1row = t1_b.reshape(1, DH)
    b2row = t2_b.reshape(1, DO)

    raw, pq0 = _prologue(h, t1_W, b1row, g0, gb0)

    zp0 = _edge_kernel(raw, pq0[:, 0], pq0[:, 1], deg2, src, dst)
    x1, pq1 = _mid(raw, zp0, g1, gb1)
    zp1 = _edge_kernel(x1, pq1[:, 0], pq1[:, 1], deg2, src, dst)
    return _epilogue(raw, zp1, t2_W, b2row)


# speedup vs baseline: 14.8187x; 1.3951x over previous
import functools

import jax
import jax.numpy as jnp
from jax import lax
from jax.experimental import pallas as pl
from jax.experimental.pallas import tpu as pltpu
from jax.experimental.pallas import tpu_sc as plsc

N = 10000
E = 320000
DH = 128
DO = 64
EPS = 0.3

NC = 2
NS = 16
NW = NC * NS
NP = 10240
RPT = NP // NS
EC = E // NC
ET = E // NW
C = 80
NCHUNK = ET // C

_mesh = plsc.VectorSubcoreMesh(
    core_axis_name="c", subcore_axis_name="s", num_cores=NC, num_subcores=NS
)


@functools.partial(
    pl.kernel,
    out_type=jax.ShapeDtypeStruct((NC, NP), jnp.float32),
    mesh=_mesh,
    scratch_types=[
        pltpu.VMEM((C,), jnp.int32),
        pltpu.VMEM((C,), jnp.float32),
        pltpu.VMEM((RPT,), jnp.float32),
        pltpu.VMEM_SHARED((NP,), jnp.float32),
    ],
    compiler_params=pltpu.CompilerParams(needs_layout_passes=False),
)
def _deg_kernel(dst_hbm, deg_out, dst_v, ones_v, zeros_v, deg_sh):
    cid = lax.axis_index("c")
    sid = lax.axis_index("s")

    def _fill(i, _):
        zeros_v[pl.ds(i * 16, 16)] = jnp.zeros((16,), jnp.float32)
        return 0

    lax.fori_loop(0, RPT // 16, _fill, 0)
    for i in range(C // 16):
        ones_v[pl.ds(i * 16, 16)] = jnp.ones((16,), jnp.float32)

    pltpu.sync_copy(zeros_v, deg_sh.at[pl.ds(sid * RPT, RPT)])
    plsc.subcore_barrier()

    base = cid * EC + sid * ET

    def _chunk(k, _):
        eo = pl.multiple_of(base + k * C, 8)
        pltpu.sync_copy(dst_hbm.at[pl.ds(eo, C)], dst_v)
        pltpu.sync_copy(ones_v, deg_sh.at[dst_v], add=True)
        return 0

    lax.fori_loop(0, NCHUNK, _chunk, 0)
    plsc.subcore_barrier()
    pltpu.sync_copy(
        deg_sh.at[pl.ds(sid * RPT, RPT)], deg_out.at[cid, pl.ds(sid * RPT, RPT)]
    )


@functools.partial(
    pl.kernel,
    out_type=jax.ShapeDtypeStruct((NC, NP, DH), jnp.float32),
    mesh=_mesh,
    scratch_types=[
        pltpu.VMEM((N,), jnp.float32),
        pltpu.VMEM((N,), jnp.float32),
        pltpu.VMEM((C,), jnp.int32),
        pltpu.VMEM((C,), jnp.int32),
        pltpu.VMEM((C,), jnp.int32),
        pltpu.VMEM((C,), jnp.int32),
        pltpu.VMEM((C, DH), jnp.float32),
        pltpu.VMEM((C, DH), jnp.float32),
        pltpu.VMEM((C,), jnp.float32),
        pltpu.VMEM_SHARED((NP, DH), jnp.float32),
        pltpu.SemaphoreType.DMA,
        pltpu.SemaphoreType.DMA,
    ],
    compiler_params=pltpu.CompilerParams(needs_layout_passes=False),
)
def _edge_kernel(
    xs_hbm, p_hbm, q_hbm, src_hbm, dst_hbm, z_out,
    p_v, q_v, src_v0, dst_v0, src_v1, dst_v1, rows_v0, rows_v1, coef_v,
    z_sh, sem0, sem1,
):
    cid = lax.axis_index("c")
    sid = lax.axis_index("s")

    pltpu.sync_copy(p_hbm, p_v)
    pltpu.sync_copy(q_hbm, q_v)

    def _zrow(i, _):
        for j in range(DH // 16):
            rows_v0[i, pl.ds(j * 16, 16)] = jnp.zeros((16,), jnp.float32)
        return 0

    lax.fori_loop(0, C, _zrow, 0)
    for i in range(RPT // C):
        pltpu.sync_copy(rows_v0, z_sh.at[pl.ds(sid * RPT + i * C, C)])
    plsc.subcore_barrier()

    base = cid * EC + sid * ET

    def _load_idx(k, sv, dv):
        eo = pl.multiple_of(base + k * C, 8)
        pltpu.sync_copy(src_hbm.at[pl.ds(eo, C)], sv)
        pltpu.sync_copy(dst_hbm.at[pl.ds(eo, C)], dv)

    def _wait_rows(rv, sem):
        pltpu.make_async_copy(xs_hbm.at[pl.ds(0, C)], rv, sem).wait()

    def _process(sv, dv, rv):
        def _gate(i, _):
            s = pl.ds(i * 16, 16)
            t = plsc.load_gather(p_v, [dv[s]]) + plsc.load_gather(q_v, [sv[s]])
            e2 = jnp.exp(t + t)
            coef_v[s] = 1.0 - 2.0 / (e2 + 1.0)
            return 0

        lax.fori_loop(0, C // 16, _gate, 0)

        def _scale(i, _):
            cvec = coef_v[pl.ds(i * 16, 16)]
            for l in range(16):
                e = i * 16 + l
                cc = cvec[l]
                for j in range(DH // 16):
                    s = pl.ds(j * 16, 16)
                    rv[e, s] = rv[e, s] * cc
            return 0

        lax.fori_loop(0, C // 16, _scale, 0)
        pltpu.sync_copy(rv, z_sh.at[dv], add=True)

    _load_idx(0, src_v0, dst_v0)
    pltpu.async_copy(xs_hbm.at[src_v0], rows_v0, sem0)

    def _pipe(j, _):
        k0 = 2 * j
        _load_idx(k0 + 1, src_v1, dst_v1)
        pltpu.async_copy(xs_hbm.at[src_v1], rows_v1, sem1)
        _wait_rows(rows_v0, sem0)
        _process(src_v0, dst_v0, rows_v0)
        _load_idx(k0 + 2, src_v0, dst_v0)
        pltpu.async_copy(xs_hbm.at[src_v0], rows_v0, sem0)
        _wait_rows(rows_v1, sem1)
        _process(src_v1, dst_v1, rows_v1)
        return 0

    lax.fori_loop(0, (NCHUNK - 1) // 2, _pipe, 0)
    _wait_rows(rows_v0, sem0)
    _process(src_v0, dst_v0, rows_v0)

    plsc.subcore_barrier()
    pltpu.sync_copy(
        z_sh.at[pl.ds(sid * RPT, RPT)], z_out.at[cid, pl.ds(sid * RPT, RPT)]
    )


_R = 1000
_GRID = N // _R


def _prologue_body(
    h_ref, w1_ref, b1_ref, g_ref, gb_ref, deg0_ref, deg1_ref,
    x_ref, xs_ref, pq_ref, dcol_ref,
):
    x = jnp.maximum(
        jnp.dot(h_ref[...], w1_ref[...], preferred_element_type=jnp.float32)
        + b1_ref[...],
        0.0,
    )
    x_ref[...] = x
    pq_ref[...] = (
        jnp.dot(x, g_ref[...], preferred_element_type=jnp.float32) + gb_ref[...]
    )
    d = lax.rsqrt(jnp.maximum(deg0_ref[...] + deg1_ref[...], 1.0))
    dcol_ref[...] = d
    xs_ref[...] = x * d


def _mid_body(raw_ref, z0_ref, z1_ref, dcol_ref, g_ref, gb_ref, xs_ref, pq_ref):
    d = dcol_ref[...]
    x = EPS * raw_ref[...] + d * (z0_ref[0] + z1_ref[0])
    xs_ref[...] = x * d
    pq_ref[...] = (
        jnp.dot(x, g_ref[...], preferred_element_type=jnp.float32) + gb_ref[...]
    )


def _epilogue_body(raw_ref, z0_ref, z1_ref, dcol_ref, w2_ref, b2_ref, out_ref):
    x = EPS * raw_ref[...] + dcol_ref[...] * (z0_ref[0] + z1_ref[0])
    o = jnp.dot(x, w2_ref[...], preferred_element_type=jnp.float32) + b2_ref[...]
    m = jnp.max(o, axis=1, keepdims=True)
    s = o - m
    out_ref[...] = s - jnp.log(jnp.sum(jnp.exp(s), axis=1, keepdims=True))


def _row_spec():
    return pl.BlockSpec((_R, DH), lambda i: (i, 0))


def _col_spec():
    return pl.BlockSpec((_R, 1), lambda i: (i, 0))


def _full_spec(shape):
    return pl.BlockSpec(shape, lambda i: tuple(0 for _ in shape))


def _z_spec(core):
    return pl.BlockSpec((1, _R, DH), lambda i, c=core: (c, i, 0))


def _prologue(h, w1, b1row, gpad, gbrow, deg0col, deg1col):
    return pl.pallas_call(
        _prologue_body,
        grid=(_GRID,),
        in_specs=[
            _row_spec(),
            _full_spec((DH, DH)),
            _full_spec((1, DH)),
            _full_spec((DH, DH)),
            _full_spec((1, DH)),
            _col_spec(),
            _col_spec(),
        ],
        out_specs=[_row_spec(), _row_spec(), _row_spec(), _col_spec()],
        out_shape=[
            jax.ShapeDtypeStruct((N, DH), jnp.float32),
            jax.ShapeDtypeStruct((N, DH), jnp.float32),
            jax.ShapeDtypeStruct((N, DH), jnp.float32),
            jax.ShapeDtypeStruct((N, 1), jnp.float32),
        ],
    )(h, w1, b1row, gpad, gbrow, deg0col, deg1col)


def _mid(raw, zp, dcol, gpad, gbrow):
    return pl.pallas_call(
        _mid_body,
        grid=(_GRID,),
        in_specs=[
            _row_spec(),
            _z_spec(0),
            _z_spec(1),
            _col_spec(),
            _full_spec((DH, DH)),
            _full_spec((1, DH)),
        ],
        out_specs=[_row_spec(), _row_spec()],
        out_shape=[
            jax.ShapeDtypeStruct((N, DH), jnp.float32),
            jax.ShapeDtypeStruct((N, DH), jnp.float32),
        ],
    )(raw, zp, zp, dcol, gpad, gbrow)


def _epilogue(raw, zp, dcol, w2, b2row):
    return pl.pallas_call(
        _epilogue_body,
        grid=(_GRID,),
        in_specs=[
            _row_spec(),
            _z_spec(0),
            _z_spec(1),
            _col_spec(),
            _full_spec((DH, DO)),
            _full_spec((1, DO)),
        ],
        out_specs=pl.BlockSpec((_R, DO), lambda i: (i, 0)),
        out_shape=jax.ShapeDtypeStruct((N, DO), jnp.float32),
    )(raw, zp, zp, dcol, w2, b2row)


def _gate_pack(gate_W, gate_b):
    g = jnp.zeros((DH, DH), jnp.float32)
    g = g.at[:, 0].set(gate_W[:DH, 0]).at[:, 1].set(gate_W[DH:, 0])
    b = jnp.zeros((1, DH), jnp.float32).at[0, 0].set(gate_b[0])
    return g, b


def kernel(h, edge_index, t1_W, t1_b, gate_W0, gate_b0, gate_W1, gate_b1, t2_W, t2_b):
    src = edge_index[0]
    dst = edge_index[1]

    deg2 = _deg_kernel(dst)

    g0, gb0 = _gate_pack(gate_W0, gate_b0)
    g1, gb1 = _gate_pack(gate_W1, gate_b1)
    b1row = t1_b.reshape(1, DH)
    b2row = t2_b.reshape(1, DO)

    raw, xs0, pq0, dcol = _prologue(
        h, t1_W, b1row, g0, gb0,
        deg2[0].reshape(NP, 1), deg2[1].reshape(NP, 1),
    )

    up0 = _edge_kernel(xs0, pq0[:, 0], pq0[:, 1], src, dst)
    xs1, pq1 = _mid(raw, up0, dcol, g1, gb1)
    up1 = _edge_kernel(xs1, pq1[:, 0], pq1[:, 1], src, dst)
    return _epilogue(raw, up1, dcol, t2_W, b2row)


# speedup vs baseline: 23.1480x; 1.5621x over previous
import functools

import jax
import jax.numpy as jnp
from jax import lax
from jax.experimental import pallas as pl
from jax.experimental.pallas import tpu as pltpu
from jax.experimental.pallas import tpu_sc as plsc

N = 10000
E = 320000
DH = 128
DO = 64
EPS = 0.3

NC = 2
NS = 16
NW = NC * NS
NP = 10240
RPT = NP // NS
EC = E // NC
ET = E // NW
C = 80
NCHUNK = ET // C

_mesh = plsc.VectorSubcoreMesh(
    core_axis_name="c", subcore_axis_name="s", num_cores=NC, num_subcores=NS
)


@functools.partial(
    pl.kernel,
    out_type=jax.ShapeDtypeStruct((NC, NP), jnp.float32),
    mesh=_mesh,
    scratch_types=[
        pltpu.VMEM((ET,), jnp.int32),
        pltpu.VMEM((C,), jnp.float32),
        pltpu.VMEM((RPT,), jnp.float32),
        pltpu.VMEM_SHARED((NP,), jnp.float32),
        pltpu.SemaphoreType.DMA,
    ],
    compiler_params=pltpu.CompilerParams(needs_layout_passes=False),
)
def _deg_kernel(dst_hbm, deg_out, dstf_v, ones_v, zeros_v, deg_sh, sem):
    cid = lax.axis_index("c")
    sid = lax.axis_index("s")

    def _fill(i, _):
        zeros_v[pl.ds(i * 16, 16)] = jnp.zeros((16,), jnp.float32)
        return 0

    lax.fori_loop(0, RPT // 16, _fill, 0)
    for i in range(C // 16):
        ones_v[pl.ds(i * 16, 16)] = jnp.ones((16,), jnp.float32)

    base = cid * EC + sid * ET
    pltpu.sync_copy(dst_hbm.at[pl.ds(base, ET)], dstf_v)
    pltpu.sync_copy(zeros_v, deg_sh.at[pl.ds(sid * RPT, RPT)])
    plsc.subcore_barrier()

    def _fire(k, _):
        pltpu.async_copy(
            ones_v, deg_sh.at[dstf_v.at[pl.ds(k * C, C)]], sem, add=True
        )
        return 0

    lax.fori_loop(0, NCHUNK, _fire, 0)

    def _drain(k, _):
        pltpu.make_async_copy(
            ones_v, deg_sh.at[dstf_v.at[pl.ds(0, C)]], sem
        ).wait()
        return 0

    lax.fori_loop(0, NCHUNK, _drain, 0)
    plsc.subcore_barrier()
    pltpu.sync_copy(
        deg_sh.at[pl.ds(sid * RPT, RPT)], deg_out.at[cid, pl.ds(sid * RPT, RPT)]
    )


@functools.partial(
    pl.kernel,
    out_type=jax.ShapeDtypeStruct((NC, NP, DH), jnp.float32),
    mesh=_mesh,
    scratch_types=[
        pltpu.VMEM((ET,), jnp.int32),
        pltpu.VMEM((ET,), jnp.int32),
        pltpu.VMEM((C, DH), jnp.float32),
        pltpu.VMEM((C, DH), jnp.float32),
        pltpu.VMEM((C,), jnp.float32),
        pltpu.VMEM((C,), jnp.float32),
        pltpu.VMEM((C,), jnp.float32),
        pltpu.VMEM((C,), jnp.float32),
        pltpu.VMEM((C,), jnp.float32),
        pltpu.VMEM_SHARED((NP, DH), jnp.float32),
        pltpu.SemaphoreType.DMA,
        pltpu.SemaphoreType.DMA,
    ],
    compiler_params=pltpu.CompilerParams(needs_layout_passes=False),
)
def _edge_kernel(
    xs_hbm, p_hbm, q_hbm, src_hbm, dst_hbm, z_out,
    srcf_v, dstf_v, rows_v0, rows_v1, pe_v0, pe_v1, qe_v0, qe_v1, coef_v,
    z_sh, sem0, sem1,
):
    cid = lax.axis_index("c")
    sid = lax.axis_index("s")

    base = cid * EC + sid * ET
    pltpu.sync_copy(src_hbm.at[pl.ds(base, ET)], srcf_v)
    pltpu.sync_copy(dst_hbm.at[pl.ds(base, ET)], dstf_v)

    def _zrow(i, _):
        for j in range(DH // 16):
            rows_v0[i, pl.ds(j * 16, 16)] = jnp.zeros((16,), jnp.float32)
        return 0

    lax.fori_loop(0, C, _zrow, 0)
    for i in range(RPT // C):
        pltpu.sync_copy(rows_v0, z_sh.at[pl.ds(sid * RPT + i * C, C)])
    plsc.subcore_barrier()

    def _fire(k, rv, pe, qe, sem):
        si = srcf_v.at[pl.ds(k * C, C)]
        di = dstf_v.at[pl.ds(k * C, C)]
        pltpu.async_copy(xs_hbm.at[si], rv, sem)
        pltpu.async_copy(p_hbm.at[di], pe, sem)
        pltpu.async_copy(q_hbm.at[si], qe, sem)

    def _wait(rv, pe, qe, sem):
        pltpu.make_async_copy(xs_hbm.at[pl.ds(0, C)], rv, sem).wait()
        pltpu.make_async_copy(p_hbm.at[pl.ds(0, C)], pe, sem).wait()
        pltpu.make_async_copy(q_hbm.at[pl.ds(0, C)], qe, sem).wait()

    def _process(k, rv, pe, qe):
        def _gate(i, _):
            s = pl.ds(i * 16, 16)
            t = pe[s] + qe[s]
            e2 = jnp.exp(t + t)
            coef_v[s] = 1.0 - 2.0 / (e2 + 1.0)
            return 0

        lax.fori_loop(0, C // 16, _gate, 0)

        def _scale(i, _):
            cvec = coef_v[pl.ds(i * 16, 16)]
            for l in range(16):
                e = i * 16 + l
                cc = cvec[l]
                for j in range(DH // 16):
                    s = pl.ds(j * 16, 16)
                    rv[e, s] = rv[e, s] * cc
            return 0

        lax.fori_loop(0, C // 16, _scale, 0)
        pltpu.sync_copy(rv, z_sh.at[dstf_v.at[pl.ds(k * C, C)]], add=True)

    _fire(0, rows_v0, pe_v0, qe_v0, sem0)
    _fire(1, rows_v1, pe_v1, qe_v1, sem1)

    def _pipe(j, _):
        k0 = 2 * j
        _wait(rows_v0, pe_v0, qe_v0, sem0)
        _process(k0, rows_v0, pe_v0, qe_v0)
        _fire(k0 + 2, rows_v0, pe_v0, qe_v0, sem0)
        _wait(rows_v1, pe_v1, qe_v1, sem1)
        _process(k0 + 1, rows_v1, pe_v1, qe_v1)
        _fire(k0 + 3, rows_v1, pe_v1, qe_v1, sem1)
        return 0

    lax.fori_loop(0, (NCHUNK - 3) // 2, _pipe, 0)
    _wait(rows_v0, pe_v0, qe_v0, sem0)
    _process(NCHUNK - 3, rows_v0, pe_v0, qe_v0)
    _fire(NCHUNK - 1, rows_v0, pe_v0, qe_v0, sem0)
    _wait(rows_v1, pe_v1, qe_v1, sem1)
    _process(NCHUNK - 2, rows_v1, pe_v1, qe_v1)
    _wait(rows_v0, pe_v0, qe_v0, sem0)
    _process(NCHUNK - 1, rows_v0, pe_v0, qe_v0)

    plsc.subcore_barrier()
    pltpu.sync_copy(
        z_sh.at[pl.ds(sid * RPT, RPT)], z_out.at[cid, pl.ds(sid * RPT, RPT)]
    )


_R = 1000
_GRID = N // _R


def _prologue_body(
    h_ref, w1_ref, b1_ref, g_ref, gb_ref, deg0_ref, deg1_ref,
    x_ref, xs_ref, pq_ref, dcol_ref,
):
    x = jnp.maximum(
        jnp.dot(h_ref[...], w1_ref[...], preferred_element_type=jnp.float32)
        + b1_ref[...],
        0.0,
    )
    x_ref[...] = x
    pq_ref[...] = (
        jnp.dot(x, g_ref[...], preferred_element_type=jnp.float32) + gb_ref[...]
    )
    d = lax.rsqrt(jnp.maximum(deg0_ref[...] + deg1_ref[...], 1.0))
    dcol_ref[...] = d
    xs_ref[...] = x * d


def _mid_body(raw_ref, z0_ref, z1_ref, dcol_ref, g_ref, gb_ref, xs_ref, pq_ref):
    d = dcol_ref[...]
    x = EPS * raw_ref[...] + d * (z0_ref[0] + z1_ref[0])
    xs_ref[...] = x * d
    pq_ref[...] = (
        jnp.dot(x, g_ref[...], preferred_element_type=jnp.float32) + gb_ref[...]
    )


def _epilogue_body(raw_ref, z0_ref, z1_ref, dcol_ref, w2_ref, b2_ref, out_ref):
    x = EPS * raw_ref[...] + dcol_ref[...] * (z0_ref[0] + z1_ref[0])
    o = jnp.dot(x, w2_ref[...], preferred_element_type=jnp.float32) + b2_ref[...]
    m = jnp.max(o, axis=1, keepdims=True)
    s = o - m
    out_ref[...] = s - jnp.log(jnp.sum(jnp.exp(s), axis=1, keepdims=True))


def _row_spec():
    return pl.BlockSpec((_R, DH), lambda i: (i, 0))


def _col_spec():
    return pl.BlockSpec((_R, 1), lambda i: (i, 0))


def _full_spec(shape):
    return pl.BlockSpec(shape, lambda i: tuple(0 for _ in shape))


def _z_spec(core):
    return pl.BlockSpec((1, _R, DH), lambda i, c=core: (c, i, 0))


def _prologue(h, w1, b1row, gpad, gbrow, deg0col, deg1col):
    return pl.pallas_call(
        _prologue_body,
        grid=(_GRID,),
        in_specs=[
            _row_spec(),
            _full_spec((DH, DH)),
            _full_spec((1, DH)),
            _full_spec((DH, DH)),
            _full_spec((1, DH)),
            _col_spec(),
            _col_spec(),
        ],
        out_specs=[_row_spec(), _row_spec(), _row_spec(), _col_spec()],
        out_shape=[
            jax.ShapeDtypeStruct((N, DH), jnp.float32),
            jax.ShapeDtypeStruct((N, DH), jnp.float32),
            jax.ShapeDtypeStruct((N, DH), jnp.float32),
            jax.ShapeDtypeStruct((N, 1), jnp.float32),
        ],
    )(h, w1, b1row, gpad, gbrow, deg0col, deg1col)


def _mid(raw, zp, dcol, gpad, gbrow):
    return pl.pallas_call(
        _mid_body,
        grid=(_GRID,),
        in_specs=[
            _row_spec(),
            _z_spec(0),
            _z_spec(1),
            _col_spec(),
            _full_spec((DH, DH)),
            _full_spec((1, DH)),
        ],
        out_specs=[_row_spec(), _row_spec()],
        out_shape=[
            jax.ShapeDtypeStruct((N, DH), jnp.float32),
            jax.ShapeDtypeStruct((N, DH), jnp.float32),
        ],
    )(raw, zp, zp, dcol, gpad, gbrow)


def _epilogue(raw, zp, dcol, w2, b2row):
    return pl.pallas_call(
        _epilogue_body,
        grid=(_GRID,),
        in_specs=[
            _row_spec(),
            _z_spec(0),
            _z_spec(1),
            _col_spec(),
            _full_spec((DH, DO)),
            _full_spec((1, DO)),
        ],
        out_specs=pl.BlockSpec((_R, DO), lambda i: (i, 0)),
        out_shape=jax.ShapeDtypeStruct((N, DO), jnp.float32),
    )(raw, zp, zp, dcol, w2, b2row)


def _gate_pack(gate_W, gate_b):
    g = jnp.zeros((DH, DH), jnp.float32)
    g = g.at[:, 0].set(gate_W[:DH, 0]).at[:, 1].set(gate_W[DH:, 0])
    b = jnp.zeros((1, DH), jnp.float32).at[0, 0].set(gate_b[0])
    return g, b


def kernel(h, edge_index, t1_W, t1_b, gate_W0, gate_b0, gate_W1, gate_b1, t2_W, t2_b):
    src = edge_index[0]
    dst = edge_index[1]

    deg2 = _deg_kernel(dst)

    g0, gb0 = _gate_pack(gate_W0, gate_b0)
    g1, gb1 = _gate_pack(gate_W1, gate_b1)
    b1row = t1_b.reshape(1, DH)
    b2row = t2_b.reshape(1, DO)

    raw, xs0, pq0, dcol = _prologue(
        h, t1_W, b1row, g0, gb0,
        deg2[0].reshape(NP, 1), deg2[1].reshape(NP, 1),
    )

    up0 = _edge_kernel(xs0, pq0[:, 0], pq0[:, 1], src, dst)
    xs1, pq1 = _mid(raw, up0, dcol, g1, gb1)
    up1 = _edge_kernel(xs1, pq1[:, 0], pq1[:, 1], src, dst)
    return _epilogue(raw, up1, dcol, t2_W, b2row)
